# Initial kernel scaffold; baseline (speedup 1.0000x reference)
import jax
import jax.numpy as jnp
from jax.experimental import pallas as pl


def kernel(x, W1, b1, g1, be1, W2, b2, g2, be2, W3, b3):
    raise NotImplementedError("write your pallas kernel here")


# speedup vs baseline: 2.5213x; 2.5213x over previous
import functools

import jax
import jax.numpy as jnp
from jax import lax
from jax.experimental import pallas as pl
from jax.experimental.pallas import tpu as pltpu
from jax.experimental.pallas import tpu_sc as plsc

B = 32
L = 4096
NHB = 128
NW = 127
FEAT_PAD = 384

GTAB_PAD = 72
LEV_OFF = 72
LUTG_OFF = 136
LUTG_PAD = 4104
AUX_N = LUTG_OFF + LUTG_PAD


def _sc_feats_body(x_hbm, aux_hbm, feats_hbm,
                   xrow, counts, gtab_v, lev_v, lutg_v, ent_s, feats_v):
    nc = 2
    row = lax.axis_index("s") * nc + lax.axis_index("c")
    pltpu.sync_copy(x_hbm.at[pl.ds(row * L, L)], xrow)
    pltpu.sync_copy(aux_hbm.at[pl.ds(0, GTAB_PAD)], gtab_v)
    pltpu.sync_copy(aux_hbm.at[pl.ds(LEV_OFF, 64)], lev_v)
    pltpu.sync_copy(aux_hbm.at[pl.ds(LUTG_OFF, LUTG_PAD)], lutg_v)

    lanes = lax.iota(jnp.int32, 16)
    zeros16i = jnp.zeros((16,), jnp.int32)
    ones16i = jnp.ones((16,), jnp.int32)
    lane32 = lanes * 32
    lane256 = lanes * 256

    def zbody(i, c):
        counts[pl.ds(i * 16, 16)] = zeros16i
        return c
    lax.fori_loop(0, (NHB * 256) // 16, zbody, 0)

    def hbody(j, c):
        g = j >> 5
        p = j & 31
        vals = plsc.load_gather(xrow, [g * 512 + p + lane32])
        vals = jnp.minimum(jnp.maximum(vals, 0), 255)
        plsc.addupdate_scatter(counts, [g * 4096 + lane256 + vals], ones16i)
        return c
    lax.fori_loop(0, 256, hbody, 0)

    def wbody(w, c):
        def ebody(k, acc):
            xv = xrow[pl.ds(w * 32 + k * 16, 16)]
            xv = jnp.minimum(jnp.maximum(xv, 0), 255)
            cnt = (plsc.load_gather(counts, [w * 256 + xv])
                   + plsc.load_gather(counts, [w * 256 + 256 + xv]))
            return acc + plsc.load_gather(gtab_v, [cnt])
        acc = lax.fori_loop(0, 4, ebody, jnp.zeros((16,), jnp.float32))
        ent_s[w] = jnp.sum(acc)
        return c
    lax.fori_loop(0, NW, wbody, 0)

    gacc = jnp.zeros((16,), jnp.float32)
    for k in range(16):
        def gbody(h, a):
            return a + counts[pl.ds(h * 256 + k * 16, 16)]
        hk = lax.fori_loop(0, NHB, gbody, zeros16i)
        feats_v[pl.ds(k * 16, 16)] = hk.astype(jnp.float32) * (1.0 / 4096.0)
        gacc = gacc + plsc.load_gather(lutg_v, [hk])
    g_ent = jnp.sum(gacc)

    for lc in range(4):
        lev = lev_v[pl.ds(lc * 16, 16)]
        def cbody(w, a):
            e = ent_s[w]
            return a + (lev >= e).astype(jnp.float32)
        acc = lax.fori_loop(0, NW, cbody, jnp.zeros((16,), jnp.float32))
        feats_v[pl.ds(256 + lc * 16, 16)] = acc / jnp.float32(NW)

    feats_v[pl.ds(320, 16)] = jnp.where(lanes == 0, g_ent, 0.0)
    zf = jnp.zeros((16,), jnp.float32)
    for k in range(3):
        feats_v[pl.ds(336 + k * 16, 16)] = zf
    pltpu.sync_copy(feats_v, feats_hbm.at[pl.ds(row * FEAT_PAD, FEAT_PAD)])


def _make_sc_feats():
    mesh = plsc.VectorSubcoreMesh(core_axis_name="c", subcore_axis_name="s")
    return pl.kernel(
        _sc_feats_body,
        mesh=mesh,
        compiler_params=pltpu.CompilerParams(needs_layout_passes=False),
        out_type=jax.ShapeDtypeStruct((B * FEAT_PAD,), jnp.float32),
        scratch_types=[
            pltpu.VMEM((L,), jnp.int32),
            pltpu.VMEM((NHB * 256,), jnp.int32),
            pltpu.VMEM((GTAB_PAD,), jnp.float32),
            pltpu.VMEM((64,), jnp.float32),
            pltpu.VMEM((LUTG_PAD,), jnp.float32),
            pltpu.SMEM((NW,), jnp.float32),
            pltpu.VMEM((FEAT_PAD,), jnp.float32),
        ],
    )


def _dot(a, b):
    return lax.dot_general(a, b, (((1,), (0,)), ((), ())),
                           precision=lax.Precision.HIGHEST,
                           preferred_element_type=jnp.float32)


def _ln_relu(h, g, b):
    mu = jnp.mean(h, axis=-1, keepdims=True)
    d = h - mu
    var = jnp.mean(d * d, axis=-1, keepdims=True)
    return jnp.maximum(d / jnp.sqrt(var + 1e-5) * g + b, 0.0)


def _mlp_body(f_ref, w1_ref, b1_ref, g1_ref, be1_ref, w2_ref, b2_ref, g2_ref,
              be2_ref, w3_ref, b3_ref, o_ref):
    h = _dot(f_ref[:], w1_ref[:]) + b1_ref[:]
    h = _ln_relu(h, g1_ref[:], be1_ref[:])
    h = _dot(h, w2_ref[:]) + b2_ref[:]
    h = _ln_relu(h, g2_ref[:], be2_ref[:])
    o_ref[:] = _dot(h, w3_ref[:]) + b3_ref[:]


def kernel(x, W1, b1, g1, be1, W2, b2, g2, be2, W3, b3):
    eps = 1e-10
    cw = jnp.arange(65, dtype=jnp.float32)
    wp = cw * (1.0 / 64.0)
    lutw = -(wp * jnp.log2(wp + eps))
    gtab = jnp.where(cw > 0, lutw / jnp.maximum(cw, 1.0), 0.0)
    gtab = jnp.pad(gtab, (0, GTAB_PAD - 65))
    levels = jnp.linspace(0.0, 8.0, 64).astype(jnp.float32)
    cg = jnp.arange(4097, dtype=jnp.float32)
    pg = cg * (1.0 / 4096.0)
    lutg = -(pg * jnp.log2(pg + eps))
    lutg = jnp.pad(lutg, (0, LUTG_PAD - 4097))
    aux = jnp.concatenate([gtab, levels, lutg])

    feats = _make_sc_feats()(x.reshape(-1).astype(jnp.int32), aux)
    feats = feats.reshape(B, FEAT_PAD)

    W1p = jnp.concatenate(
        [W1[0:256], W1[257:321], W1[256:257],
         jnp.zeros((FEAT_PAD - 321, W1.shape[1]), jnp.float32)], axis=0)

    out = pl.pallas_call(
        _mlp_body,
        out_shape=jax.ShapeDtypeStruct((B, W3.shape[1]), jnp.float32),
    )(feats, W1p, b1.reshape(1, -1), g1.reshape(1, -1), be1.reshape(1, -1),
      W2, b2.reshape(1, -1), g2.reshape(1, -1), be2.reshape(1, -1),
      W3, b3.reshape(1, -1))
    return out
